# baseline (speedup 1.0000x reference)
import functools

import jax
import jax.numpy as jnp
from jax import lax
from jax.experimental import pallas as pl
from jax.experimental.pallas import tpu as pltpu
from jax.experimental.pallas import tpu_sc as plsc

NC = 2
NS = 16
LANES = 16


def _sc_main(pred_hbm, target_hbm, idx_hbm, it_hbm, out_hbm,
             pbuf0, pbuf1, tbuf0, tbuf1, idx_v, bcnt_v, gmap_v, gsum_v,
             it_v, cacc_v, row2_v, acc8_v, out_v, shared_gsum, shared_acc,
             sp0, sp1, st0, st1,
             *, B, N, D, G, T, TOK, CH, NCH):
    c = lax.axis_index("c")
    s = lax.axis_index("s")
    wid = c * NS + s
    b = wid // 2
    pos_base = (wid % 2) * TOK
    lanes = lax.iota(jnp.int32, LANES)
    zero16i = jnp.zeros((LANES,), jnp.int32)
    zero16f = jnp.zeros((LANES,), jnp.float32)

    pltpu.sync_copy(idx_hbm.at[b], idx_v)

    def _zb(i, carry):
        bcnt_v[pl.ds(i * LANES, LANES)] = zero16i
        return carry
    lax.fori_loop(0, TOK // LANES, _zb, 0)

    ones16i = jnp.ones((LANES,), jnp.int32)
    for v in range((G + LANES) // LANES):
        j = v * LANES + lanes
        bv = idx_v[pl.ds(v * LANES, LANES)]
        rel = bv - pos_base
        valid = (j <= G) & (rel < TOK)
        tgt = jnp.maximum(rel, 0)
        plsc.addupdate_scatter(bcnt_v, [tgt], ones16i, mask=valid)

    def _cs(i, carry):
        x = bcnt_v[pl.ds(i * LANES, LANES)]
        cs = plsc.cumsum(x) + carry
        gmap_v[pl.ds(i * LANES, LANES)] = cs - 1
        return carry + jnp.sum(x)
    lax.fori_loop(0, TOK // LANES, _cs, jnp.int32(0))

    for k in range(128 // LANES):
        gsum_v[pl.ds(k * LANES, LANES)] = zero16f

    CHW = CH * D

    def _issue(ch, pb, tb, sp, st):
        off = (b * N + pos_base + ch * CH) * D
        pltpu.async_copy(pred_hbm.at[pl.ds(off, CHW)], pb, sp)
        pltpu.async_copy(target_hbm.at[pl.ds(off, CHW)], tb, st)

    def _wait(pb, tb, sp, st):
        pltpu.make_async_copy(pred_hbm.at[pl.ds(0, CHW)], pb, sp).wait()
        pltpu.make_async_copy(target_hbm.at[pl.ds(0, CHW)], tb, st).wait()

    def _compute(pb, tb, ch):
        def _tg(tg, carry):
            base_idx = (tg * LANES + lanes) * D

            def _d8(i8, acc):
                for k in range(8):
                    idx = base_idx + (i8 * 8 + k)
                    pv = plsc.load_gather(pb, [idx])
                    tv = plsc.load_gather(tb, [idx])
                    dv = pv - tv
                    acc = acc + dv * dv
                return acc
            te = lax.fori_loop(0, D // 8, _d8, zero16f)
            te = te * (1.0 / D)
            g = gmap_v[pl.ds(ch * CH + tg * LANES, LANES)]
            gvalid = (g >= 0) & (g < G)
            gsafe = jnp.minimum(jnp.maximum(g, 0), G - 1)
            plsc.addupdate_scatter(gsum_v, [gsafe], te, mask=gvalid)
            return carry
        lax.fori_loop(0, CH // LANES, _tg, 0)

    _issue(0, pbuf0, tbuf0, sp0, st0)
    _issue(1, pbuf1, tbuf1, sp1, st1)

    def _pair(i, carry):
        ch0 = 2 * i
        _wait(pbuf0, tbuf0, sp0, st0)
        _compute(pbuf0, tbuf0, ch0)

        @pl.when(ch0 + 2 < NCH)
        def _():
            _issue(ch0 + 2, pbuf0, tbuf0, sp0, st0)

        _wait(pbuf1, tbuf1, sp1, st1)
        _compute(pbuf1, tbuf1, ch0 + 1)

        @pl.when(ch0 + 3 < NCH)
        def _():
            _issue(ch0 + 3, pbuf1, tbuf1, sp1, st1)
        return carry
    lax.fori_loop(0, NCH // 2, _pair, 0)

    pltpu.sync_copy(gsum_v, shared_gsum.at[pl.ds(s * 128, 128)])
    plsc.subcore_barrier()

    HB = B // NC

    @pl.when(s < HB)
    def _():
        bg = c * HB + s
        pltpu.sync_copy(shared_gsum.at[pl.ds((2 * s) * 128, 256)], row2_v)
        pltpu.sync_copy(idx_hbm.at[bg], idx_v)
        pltpu.sync_copy(it_hbm.at[bg], it_v)
        cacc_v[pl.ds(0, LANES)] = zero16f
        cacc_v[pl.ds(LANES, LANES)] = zero16f
        ones16f = jnp.ones((LANES,), jnp.float32)
        for j in range(G // LANES):
            gl = j * LANES + lanes
            st_ = plsc.load_gather(idx_v, [gl])
            en_ = plsc.load_gather(idx_v, [gl + 1])
            cnt = (en_ - st_).astype(jnp.float32)
            gs = row2_v[pl.ds(j * LANES, LANES)] + row2_v[pl.ds(128 + j * LANES, LANES)]
            ge = gs / jnp.maximum(cnt, 1.0)
            base2 = gl * T
            m0 = plsc.load_gather(it_v, [base2])
            am = zero16i
            for tt in range(1, T):
                col = plsc.load_gather(it_v, [base2 + tt])
                better = col > m0
                am = jnp.where(better, tt, am)
                m0 = jnp.where(better, col, m0)
            plsc.addupdate_scatter(cacc_v, [am], ge)
            plsc.addupdate_scatter(cacc_v, [am + T], ones16f)
        pltpu.sync_copy(cacc_v.at[pl.ds(0, 2 * T)],
                        shared_acc.at[pl.ds(s * (2 * T), 2 * T)])

    plsc.subcore_barrier()

    @pl.when(s == 0)
    def _():
        pltpu.sync_copy(shared_acc, acc8_v)
        ts = zero16f
        tc = zero16f
        for i in range(HB):
            ts = ts + acc8_v[pl.ds(i * (2 * T), T)]
            tc = tc + acc8_v[pl.ds(i * (2 * T) + T, T)]
        out_v[pl.ds(0, T)] = ts
        out_v[pl.ds(T, T)] = tc
        pltpu.sync_copy(out_v, out_hbm.at[c])


def _sc_combine(part_hbm, out_hbm, pv, ov, *, T):
    c = lax.axis_index("c")
    s = lax.axis_index("s")

    @pl.when((c == 0) & (s == 0))
    def _():
        pltpu.sync_copy(part_hbm, pv)
        ts = pv[pl.ds(0, T)] + pv[pl.ds(2 * T, T)]
        tc = pv[pl.ds(T, T)] + pv[pl.ds(3 * T, T)]
        per = jnp.where(tc > 0, ts / jnp.maximum(tc, 1.0), 0.0)
        ov[...] = per
        pltpu.sync_copy(ov, out_hbm)


def kernel(pred, target, indices, indices_type, type_names):
    B, N, D = pred.shape
    G = indices.shape[1] - 1
    T = indices_type.shape[2]
    NT = NC * NS
    TOK = B * N // NT
    CH = 128
    NCH = TOK // CH

    pred1 = pred.reshape(B * N * D)
    target1 = target.reshape(B * N * D)
    idx_pad = jnp.pad(indices, ((0, 0), (0, 128 - (G + 1))))
    it2 = indices_type.reshape(B, G * T)

    mesh = plsc.VectorSubcoreMesh(core_axis_name="c", subcore_axis_name="s")
    params = pltpu.CompilerParams(needs_layout_passes=False)

    main = functools.partial(
        pl.kernel,
        out_type=jax.ShapeDtypeStruct((NC, 2 * T), jnp.float32),
        mesh=mesh,
        compiler_params=params,
        scratch_types=[
            pltpu.VMEM((CH * D,), jnp.float32),
            pltpu.VMEM((CH * D,), jnp.float32),
            pltpu.VMEM((CH * D,), jnp.float32),
            pltpu.VMEM((CH * D,), jnp.float32),
            pltpu.VMEM((128,), jnp.int32),
            pltpu.VMEM((TOK,), jnp.int32),
            pltpu.VMEM((TOK,), jnp.int32),
            pltpu.VMEM((128,), jnp.float32),
            pltpu.VMEM((G * T,), jnp.float32),
            pltpu.VMEM((128,), jnp.float32),
            pltpu.VMEM((256,), jnp.float32),
            pltpu.VMEM((NS // 2 * 2 * T,), jnp.float32),
            pltpu.VMEM((2 * T,), jnp.float32),
            pltpu.VMEM_SHARED((NS * 128,), jnp.float32),
            pltpu.VMEM_SHARED((NS // 2 * 2 * T,), jnp.float32),
            pltpu.SemaphoreType.DMA,
            pltpu.SemaphoreType.DMA,
            pltpu.SemaphoreType.DMA,
            pltpu.SemaphoreType.DMA,
        ],
    )(functools.partial(_sc_main, B=B, N=N, D=D, G=G, T=T,
                        TOK=TOK, CH=CH, NCH=NCH))
    partials = main(pred1, target1, idx_pad, it2)
    part1 = partials.reshape(NC * 2 * T)

    comb = functools.partial(
        pl.kernel,
        out_type=jax.ShapeDtypeStruct((T,), jnp.float32),
        mesh=mesh,
        compiler_params=params,
        scratch_types=[
            pltpu.VMEM((NC * 2 * T,), jnp.float32),
            pltpu.VMEM((T,), jnp.float32),
        ],
    )(functools.partial(_sc_combine, T=T))
    return comb(part1)


# speedup vs baseline: 3.2361x; 3.2361x over previous
import functools

import jax
import jax.numpy as jnp
from jax import lax
from jax.experimental import pallas as pl
from jax.experimental.pallas import tpu as pltpu
from jax.experimental.pallas import tpu_sc as plsc

NC = 2
NS = 16
LANES = 16


def _sc_main(pred_hbm, target_hbm, idx_hbm, it_hbm, out_hbm,
             pbuf0, pbuf1, tbuf0, tbuf1, idx_v, bcnt_v, gmap_v, gsum_v,
             it_v, cacc_v, row2_v, acc8_v, out_v, shared_gsum, shared_acc,
             sp0, sp1, st0, st1,
             *, B, N, D, G, T, TOK, CH, NCH):
    c = lax.axis_index("c")
    s = lax.axis_index("s")
    wid = c * NS + s
    b = wid // 2
    pos_base = (wid % 2) * TOK
    lanes = lax.iota(jnp.int32, LANES)
    zero16i = jnp.zeros((LANES,), jnp.int32)
    zero16f = jnp.zeros((LANES,), jnp.float32)

    pltpu.sync_copy(idx_hbm.at[b], idx_v)

    def _zb(i, carry):
        bcnt_v[pl.ds(i * LANES, LANES)] = zero16i
        return carry
    lax.fori_loop(0, TOK // LANES, _zb, 0)

    ones16i = jnp.ones((LANES,), jnp.int32)
    for v in range((G + LANES) // LANES):
        j = v * LANES + lanes
        bv = idx_v[pl.ds(v * LANES, LANES)]
        rel = bv - pos_base
        valid = (j <= G) & (rel < TOK)
        tgt = jnp.maximum(rel, 0)
        plsc.addupdate_scatter(bcnt_v, [tgt], ones16i, mask=valid)

    def _cs(i, carry):
        x = bcnt_v[pl.ds(i * LANES, LANES)]
        cs = plsc.cumsum(x) + carry
        gmap_v[pl.ds(i * LANES, LANES)] = cs - 1
        return carry + jnp.sum(x)
    lax.fori_loop(0, TOK // LANES, _cs, jnp.int32(0))

    for k in range(128 // LANES):
        gsum_v[pl.ds(k * LANES, LANES)] = zero16f

    CHW = CH * D

    def _issue(ch, pb, tb, sp, st):
        off = (b * N + pos_base + ch * CH) * D
        pltpu.async_copy(pred_hbm.at[pl.ds(off, CHW)], pb, sp)
        pltpu.async_copy(target_hbm.at[pl.ds(off, CHW)], tb, st)

    def _wait(pb, tb, sp, st):
        pltpu.make_async_copy(pred_hbm.at[pl.ds(0, CHW)], pb, sp).wait()
        pltpu.make_async_copy(target_hbm.at[pl.ds(0, CHW)], tb, st).wait()

    shuf_dn = lax.GatherDimensionNumbers(
        offset_dims=(), collapsed_slice_dims=(0,), start_index_map=(0,))

    def _shuf(v, idx):
        return lax.gather(v, idx[:, None], shuf_dn, slice_sizes=(1,),
                          mode=lax.GatherScatterMode.PROMISE_IN_BOUNDS)

    xor_idx = {o: lanes ^ o for o in (8, 4, 2, 1)}
    xor_msk = {o: (lanes & o) == 0 for o in (8, 4, 2, 1)}

    def _compute(pb, tb, ch):
        def _tg(tg, carry):
            a = []
            for t in range(LANES):
                off = (tg * LANES + t) * D
                acc = None
                for v in range(D // LANES):
                    pv = pb[pl.ds(off + v * LANES, LANES)]
                    tv = tb[pl.ds(off + v * LANES, LANES)]
                    dv = pv - tv
                    sq = dv * dv
                    acc = sq if acc is None else acc + sq
                a.append(acc)
            for o in (8, 4, 2, 1):
                half = len(a) // 2
                a = [jnp.where(xor_msk[o],
                               a[k] + _shuf(a[k], xor_idx[o]),
                               a[k + half] + _shuf(a[k + half], xor_idx[o]))
                     for k in range(half)]
            te = a[0] * (1.0 / D)
            g = gmap_v[pl.ds(ch * CH + tg * LANES, LANES)]
            gvalid = (g >= 0) & (g < G)
            gsafe = jnp.minimum(jnp.maximum(g, 0), G - 1)
            plsc.addupdate_scatter(gsum_v, [gsafe], te, mask=gvalid)
            return carry
        lax.fori_loop(0, CH // LANES, _tg, 0)

    _issue(0, pbuf0, tbuf0, sp0, st0)
    _issue(1, pbuf1, tbuf1, sp1, st1)

    def _pair(i, carry):
        ch0 = 2 * i
        _wait(pbuf0, tbuf0, sp0, st0)
        _compute(pbuf0, tbuf0, ch0)

        @pl.when(ch0 + 2 < NCH)
        def _():
            _issue(ch0 + 2, pbuf0, tbuf0, sp0, st0)

        _wait(pbuf1, tbuf1, sp1, st1)
        _compute(pbuf1, tbuf1, ch0 + 1)

        @pl.when(ch0 + 3 < NCH)
        def _():
            _issue(ch0 + 3, pbuf1, tbuf1, sp1, st1)
        return carry
    lax.fori_loop(0, NCH // 2, _pair, 0)

    pltpu.sync_copy(gsum_v, shared_gsum.at[pl.ds(s * 128, 128)])
    plsc.subcore_barrier()

    HB = B // NC

    @pl.when(s < HB)
    def _():
        bg = c * HB + s
        pltpu.sync_copy(shared_gsum.at[pl.ds((2 * s) * 128, 256)], row2_v)
        pltpu.sync_copy(idx_hbm.at[bg], idx_v)
        pltpu.sync_copy(it_hbm.at[bg], it_v)
        cacc_v[pl.ds(0, LANES)] = zero16f
        cacc_v[pl.ds(LANES, LANES)] = zero16f
        ones16f = jnp.ones((LANES,), jnp.float32)
        for j in range(G // LANES):
            gl = j * LANES + lanes
            st_ = plsc.load_gather(idx_v, [gl])
            en_ = plsc.load_gather(idx_v, [gl + 1])
            cnt = (en_ - st_).astype(jnp.float32)
            gs = row2_v[pl.ds(j * LANES, LANES)] + row2_v[pl.ds(128 + j * LANES, LANES)]
            ge = gs / jnp.maximum(cnt, 1.0)
            base2 = gl * T
            m0 = plsc.load_gather(it_v, [base2])
            am = zero16i
            for tt in range(1, T):
                col = plsc.load_gather(it_v, [base2 + tt])
                better = col > m0
                am = jnp.where(better, tt, am)
                m0 = jnp.where(better, col, m0)
            plsc.addupdate_scatter(cacc_v, [am], ge)
            plsc.addupdate_scatter(cacc_v, [am + T], ones16f)
        pltpu.sync_copy(cacc_v.at[pl.ds(0, 2 * T)],
                        shared_acc.at[pl.ds(s * (2 * T), 2 * T)])

    plsc.subcore_barrier()

    @pl.when(s == 0)
    def _():
        pltpu.sync_copy(shared_acc, acc8_v)
        ts = zero16f
        tc = zero16f
        for i in range(HB):
            ts = ts + acc8_v[pl.ds(i * (2 * T), T)]
            tc = tc + acc8_v[pl.ds(i * (2 * T) + T, T)]
        out_v[pl.ds(0, T)] = ts
        out_v[pl.ds(T, T)] = tc
        pltpu.sync_copy(out_v, out_hbm.at[c])


def _sc_combine(part_hbm, out_hbm, pv, ov, *, T):
    c = lax.axis_index("c")
    s = lax.axis_index("s")

    @pl.when((c == 0) & (s == 0))
    def _():
        pltpu.sync_copy(part_hbm, pv)
        ts = pv[pl.ds(0, T)] + pv[pl.ds(2 * T, T)]
        tc = pv[pl.ds(T, T)] + pv[pl.ds(3 * T, T)]
        per = jnp.where(tc > 0, ts / jnp.maximum(tc, 1.0), 0.0)
        ov[...] = per
        pltpu.sync_copy(ov, out_hbm)


def kernel(pred, target, indices, indices_type, type_names):
    B, N, D = pred.shape
    G = indices.shape[1] - 1
    T = indices_type.shape[2]
    NT = NC * NS
    TOK = B * N // NT
    CH = 128
    NCH = TOK // CH

    pred1 = pred.reshape(B * N * D)
    target1 = target.reshape(B * N * D)
    idx_pad = jnp.pad(indices, ((0, 0), (0, 128 - (G + 1))))
    it2 = indices_type.reshape(B, G * T)

    mesh = plsc.VectorSubcoreMesh(core_axis_name="c", subcore_axis_name="s")
    params = pltpu.CompilerParams(needs_layout_passes=False)

    main = functools.partial(
        pl.kernel,
        out_type=jax.ShapeDtypeStruct((NC, 2 * T), jnp.float32),
        mesh=mesh,
        compiler_params=params,
        scratch_types=[
            pltpu.VMEM((CH * D,), jnp.float32),
            pltpu.VMEM((CH * D,), jnp.float32),
            pltpu.VMEM((CH * D,), jnp.float32),
            pltpu.VMEM((CH * D,), jnp.float32),
            pltpu.VMEM((128,), jnp.int32),
            pltpu.VMEM((TOK,), jnp.int32),
            pltpu.VMEM((TOK,), jnp.int32),
            pltpu.VMEM((128,), jnp.float32),
            pltpu.VMEM((G * T,), jnp.float32),
            pltpu.VMEM((128,), jnp.float32),
            pltpu.VMEM((256,), jnp.float32),
            pltpu.VMEM((NS // 2 * 2 * T,), jnp.float32),
            pltpu.VMEM((2 * T,), jnp.float32),
            pltpu.VMEM_SHARED((NS * 128,), jnp.float32),
            pltpu.VMEM_SHARED((NS // 2 * 2 * T,), jnp.float32),
            pltpu.SemaphoreType.DMA,
            pltpu.SemaphoreType.DMA,
            pltpu.SemaphoreType.DMA,
            pltpu.SemaphoreType.DMA,
        ],
    )(functools.partial(_sc_main, B=B, N=N, D=D, G=G, T=T,
                        TOK=TOK, CH=CH, NCH=NCH))
    partials = main(pred1, target1, idx_pad, it2)
    part1 = partials.reshape(NC * 2 * T)

    comb = functools.partial(
        pl.kernel,
        out_type=jax.ShapeDtypeStruct((T,), jnp.float32),
        mesh=mesh,
        compiler_params=params,
        scratch_types=[
            pltpu.VMEM((NC * 2 * T,), jnp.float32),
            pltpu.VMEM((T,), jnp.float32),
        ],
    )(functools.partial(_sc_combine, T=T))
    return comb(part1)
